# baseline (speedup 1.0000x reference)
import functools

import jax
import jax.numpy as jnp
from jax import lax
from jax.experimental import pallas as pl
from jax.experimental.pallas import tpu as pltpu
from jax.experimental.pallas import tpu_sc as plsc

NC = 2
NT = 16
CHUNK = 128
L = 16


def _mesh():
    return plsc.VectorSubcoreMesh(
        core_axis_name="c", subcore_axis_name="s", num_cores=NC)


_SC_PARAMS = pltpu.CompilerParams(use_tc_tiling_on_sc=False)


def _deg_body(nch, dst4, out, dst_v, ones_v, acc, *, rpt):
    cid = lax.axis_index("c")
    w = lax.axis_index("s")
    pltpu.sync_copy(dst4.at[cid, w], dst_v)

    def fill(r, _):
        ones_v[r, :] = jnp.zeros((L,), jnp.float32)
        return 0

    lax.fori_loop(0, CHUNK, fill, 0)
    for ch in range(rpt // CHUNK):
        pltpu.sync_copy(ones_v, acc.at[pl.ds(w * rpt + ch * CHUNK, CHUNK)])

    def fill1(r, _):
        ones_v[r, :] = jnp.ones((L,), jnp.float32)
        return 0

    lax.fori_loop(0, CHUNK, fill1, 0)
    plsc.subcore_barrier()

    def edge_chunk(j, _):
        pltpu.sync_copy(ones_v, acc.at[dst_v.at[j]], add=True)
        return 0

    lax.fori_loop(0, nch, edge_chunk, 0)
    plsc.subcore_barrier()
    pltpu.sync_copy(acc.at[pl.ds(w * rpt, rpt)],
                    out.at[cid, pl.ds(w * rpt, rpt)])


def _make_deg_kernel(npad, nch):
    rpt = npad // NT
    body = functools.partial(_deg_body, nch, rpt=rpt)
    return pl.kernel(
        body,
        out_type=jax.ShapeDtypeStruct((NC, npad, L), jnp.float32),
        mesh=_mesh(),
        scratch_types=[
            pltpu.VMEM((nch, CHUNK), jnp.int32),
            pltpu.VMEM((CHUNK, L), jnp.float32),
            pltpu.VMEM_SHARED((npad, L), jnp.float32),
        ],
        compiler_params=_SC_PARAMS,
    )


def _prop_body(nch, m_in, src4, dst4, p_out,
               src_v, dst_v, g0, g1, s0, s1, acc, *, rpt):
    cid = lax.axis_index("c")
    w = lax.axis_index("s")
    pltpu.sync_copy(src4.at[cid, w], src_v)
    pltpu.sync_copy(dst4.at[cid, w], dst_v)

    @pl.when(cid == 0)
    def _():
        pltpu.sync_copy(m_in.at[pl.ds(w * rpt, rpt)],
                        acc.at[pl.ds(w * rpt, rpt)])

    @pl.when(cid == 1)
    def _():
        def fill(r, _):
            g0[r, :] = jnp.zeros((64,), jnp.float32)
            return 0

        lax.fori_loop(0, CHUNK, fill, 0)
        for ch in range(rpt // CHUNK):
            pltpu.sync_copy(g0, acc.at[pl.ds(w * rpt + ch * CHUNK, CHUNK)])

    plsc.subcore_barrier()

    def edge_block(i, _):
        j = 2 * i
        d0 = pltpu.async_copy(m_in.at[src_v.at[j]], g0, s0)
        d1 = pltpu.async_copy(m_in.at[src_v.at[j + 1]], g1, s1)
        d0.wait()
        pltpu.sync_copy(g0, acc.at[dst_v.at[j]], add=True)
        d1.wait()
        pltpu.sync_copy(g1, acc.at[dst_v.at[j + 1]], add=True)
        return 0

    lax.fori_loop(0, nch // 2, edge_block, 0)
    plsc.subcore_barrier()
    pltpu.sync_copy(acc.at[pl.ds(w * rpt, rpt)],
                    p_out.at[cid, pl.ds(w * rpt, rpt)])


def _make_prop_kernel(npad, nch):
    rpt = npad // NT
    body = functools.partial(_prop_body, nch, rpt=rpt)
    return pl.kernel(
        body,
        out_type=jax.ShapeDtypeStruct((NC, npad, 64), jnp.float32),
        mesh=_mesh(),
        scratch_types=[
            pltpu.VMEM((nch, CHUNK), jnp.int32),
            pltpu.VMEM((nch, CHUNK), jnp.int32),
            pltpu.VMEM((CHUNK, 64), jnp.float32),
            pltpu.VMEM((CHUNK, 64), jnp.float32),
            pltpu.SemaphoreType.DMA,
            pltpu.SemaphoreType.DMA,
            pltpu.VMEM_SHARED((npad, 64), jnp.float32),
        ],
        compiler_params=_SC_PARAMS,
    )


def _enc_body(x_ref, w1_ref, b1_ref, w2_ref, d16_ref, u_ref, c64_ref):
    h = jnp.dot(x_ref[...], w1_ref[...],
                preferred_element_type=jnp.float32) + b1_ref[...]
    h = jnp.maximum(h, 0.0)
    g = jnp.dot(h, w2_ref[...], preferred_element_type=jnp.float32)
    deg = d16_ref[0, :, 0:1] + d16_ref[1, :, 0:1] + 1.0
    u_ref[...] = g * (1.0 / jnp.sqrt(deg))
    c64_ref[...] = jnp.broadcast_to(1.0 / deg, g.shape)


def _comb_body(acoef, bcoef, p_ref, c_ref, b_ref, o_ref):
    t = p_ref[0] + p_ref[1]
    o_ref[...] = acoef * c_ref[...] * t + bcoef * b_ref[...]


def _fin_body(p_ref, c_ref, m0_ref, b2_ref, o_ref):
    c = c_ref[...]
    m = 0.9 * c * (p_ref[0] + p_ref[1]) + 0.1 * m0_ref[...]
    z = m * jnp.sqrt(1.0 / c)
    a = z + b2_ref[...]
    mx = jnp.max(a, axis=1, keepdims=True)
    e = jnp.exp(a - mx)
    s = jnp.sum(e, axis=1, keepdims=True)
    o_ref[...] = (a - mx) - jnp.log(s)


@jax.jit
def kernel(x, edge_index, W1, b1, W2, b2):
    n, f_in = x.shape
    hid = W1.shape[1]
    cls = W2.shape[1]
    e = edge_index.shape[1]

    nw = NC * NT
    per_w = -(-e // (nw * 2 * CHUNK)) * 2 * CHUNK
    ep = per_w * nw
    nch = per_w // CHUNK
    npad = -(-n // (NT * CHUNK)) * NT * CHUNK

    src = edge_index[0]
    dst = edge_index[1]
    pad = ep - e
    srcp = jnp.concatenate([src, jnp.zeros((pad,), jnp.int32)])
    dstp = jnp.concatenate([dst, jnp.full((pad,), npad - 1, jnp.int32)])
    src4 = srcp.reshape(NC, NT, nch, CHUNK)
    dst4 = dstp.reshape(NC, NT, nch, CHUNK)

    deg16 = _make_deg_kernel(npad, nch)(dst4)

    bn = 400
    grid = (n // bn,)
    u, c64 = pl.pallas_call(
        _enc_body,
        grid=grid,
        in_specs=[
            pl.BlockSpec((bn, f_in), lambda i: (i, 0)),
            pl.BlockSpec((f_in, hid), lambda i: (0, 0)),
            pl.BlockSpec((1, hid), lambda i: (0, 0)),
            pl.BlockSpec((hid, cls), lambda i: (0, 0)),
            pl.BlockSpec((NC, bn, L), lambda i: (0, i, 0)),
        ],
        out_specs=[
            pl.BlockSpec((bn, cls), lambda i: (i, 0)),
            pl.BlockSpec((bn, cls), lambda i: (i, 0)),
        ],
        out_shape=[
            jax.ShapeDtypeStruct((n, cls), jnp.float32),
            jax.ShapeDtypeStruct((n, cls), jnp.float32),
        ],
    )(x, W1, b1.reshape(1, hid), W2, deg16)

    u = jnp.pad(u, ((0, npad - n), (0, 0)))
    c64 = jnp.pad(c64, ((0, npad - n), (0, 0)))

    prop = _make_prop_kernel(npad, nch)

    bnp = 512
    gridp = (npad // bnp,)

    def comb(acoef, bcoef, p, base):
        return pl.pallas_call(
            functools.partial(_comb_body, acoef, bcoef),
            grid=gridp,
            in_specs=[
                pl.BlockSpec((NC, bnp, cls), lambda i: (0, i, 0)),
                pl.BlockSpec((bnp, cls), lambda i: (i, 0)),
                pl.BlockSpec((bnp, cls), lambda i: (i, 0)),
            ],
            out_specs=pl.BlockSpec((bnp, cls), lambda i: (i, 0)),
            out_shape=jax.ShapeDtypeStruct((npad, cls), jnp.float32),
        )(p, c64, base)

    p = prop(u, src4, dst4)
    m0 = comb(-1.0 / 3.0, 1.0, p, u)
    m = m0
    for _ in range(9):
        p = prop(m, src4, dst4)
        m = comb(0.9, 0.1, p, m0)
    p = prop(m, src4, dst4)

    out = pl.pallas_call(
        _fin_body,
        grid=grid,
        in_specs=[
            pl.BlockSpec((NC, bn, cls), lambda i: (0, i, 0)),
            pl.BlockSpec((bn, cls), lambda i: (i, 0)),
            pl.BlockSpec((bn, cls), lambda i: (i, 0)),
            pl.BlockSpec((1, cls), lambda i: (0, 0)),
        ],
        out_specs=pl.BlockSpec((bn, cls), lambda i: (i, 0)),
        out_shape=jax.ShapeDtypeStruct((n, cls), jnp.float32),
    )(p, c64, m0, b2.reshape(1, cls))
    return out


# speedup vs baseline: 1.8224x; 1.8224x over previous
import functools

import jax
import jax.numpy as jnp
from jax import lax
from jax.experimental import pallas as pl
from jax.experimental.pallas import tpu as pltpu
from jax.experimental.pallas import tpu_sc as plsc

NC = 2
NT = 16
CHUNK = 128
L = 16


def _mesh():
    return plsc.VectorSubcoreMesh(
        core_axis_name="c", subcore_axis_name="s", num_cores=NC)


_SC_PARAMS = pltpu.CompilerParams(use_tc_tiling_on_sc=False)


def _deg_body(nch, dst4, out, dst_v, ones_v, acc, *, rpt):
    cid = lax.axis_index("c")
    w = lax.axis_index("s")
    pltpu.sync_copy(dst4.at[cid, w], dst_v)

    def fill(r, _):
        ones_v[r, :] = jnp.zeros((L,), jnp.float32)
        return 0

    lax.fori_loop(0, CHUNK, fill, 0)
    for ch in range(rpt // CHUNK):
        pltpu.sync_copy(ones_v, acc.at[pl.ds(w * rpt + ch * CHUNK, CHUNK)])

    def fill1(r, _):
        ones_v[r, :] = jnp.ones((L,), jnp.float32)
        return 0

    lax.fori_loop(0, CHUNK, fill1, 0)
    plsc.subcore_barrier()

    def edge_chunk(j, _):
        pltpu.sync_copy(ones_v, acc.at[dst_v.at[j]], add=True)
        return 0

    lax.fori_loop(0, nch, edge_chunk, 0)
    plsc.subcore_barrier()
    pltpu.sync_copy(acc.at[pl.ds(w * rpt, rpt)],
                    out.at[cid, pl.ds(w * rpt, rpt)])


def _make_deg_kernel(npad, nch):
    rpt = npad // NT
    body = functools.partial(_deg_body, nch, rpt=rpt)
    return pl.kernel(
        body,
        out_type=jax.ShapeDtypeStruct((NC, npad, L), jnp.float32),
        mesh=_mesh(),
        scratch_types=[
            pltpu.VMEM((nch, CHUNK), jnp.int32),
            pltpu.VMEM((CHUNK, L), jnp.float32),
            pltpu.VMEM_SHARED((npad, L), jnp.float32),
        ],
        compiler_params=_SC_PARAMS,
    )


def _prop_body(nch, m_in, src4, dst4, p_out,
               src_v, dst_v, g0, g1, s0, s1, acc, m_sp, *, rpt):
    cid = lax.axis_index("c")
    w = lax.axis_index("s")
    pltpu.sync_copy(src4.at[cid, w], src_v)
    pltpu.sync_copy(dst4.at[cid, w], dst_v)

    pltpu.sync_copy(m_in.at[pl.ds(w * rpt, rpt)],
                    m_sp.at[pl.ds(w * rpt, rpt)])

    @pl.when(cid == 0)
    def _():
        pltpu.sync_copy(m_in.at[pl.ds(w * rpt, rpt)],
                        acc.at[pl.ds(w * rpt, rpt)])

    @pl.when(cid == 1)
    def _():
        def fill(r, _):
            g0[r, :] = jnp.zeros((64,), jnp.float32)
            return 0

        lax.fori_loop(0, CHUNK, fill, 0)
        for ch in range(rpt // CHUNK):
            pltpu.sync_copy(g0, acc.at[pl.ds(w * rpt + ch * CHUNK, CHUNK)])

    plsc.subcore_barrier()

    def edge_block(i, _):
        j = 2 * i
        d0 = pltpu.async_copy(m_sp.at[src_v.at[j]], g0, s0)
        d1 = pltpu.async_copy(m_sp.at[src_v.at[j + 1]], g1, s1)
        d0.wait()
        pltpu.sync_copy(g0, acc.at[dst_v.at[j]], add=True)
        d1.wait()
        pltpu.sync_copy(g1, acc.at[dst_v.at[j + 1]], add=True)
        return 0

    lax.fori_loop(0, nch // 2, edge_block, 0)
    plsc.subcore_barrier()
    pltpu.sync_copy(acc.at[pl.ds(w * rpt, rpt)],
                    p_out.at[cid, pl.ds(w * rpt, rpt)])


def _make_prop_kernel(npad, nch):
    rpt = npad // NT
    body = functools.partial(_prop_body, nch, rpt=rpt)
    return pl.kernel(
        body,
        out_type=jax.ShapeDtypeStruct((NC, npad, 64), jnp.float32),
        mesh=_mesh(),
        scratch_types=[
            pltpu.VMEM((nch, CHUNK), jnp.int32),
            pltpu.VMEM((nch, CHUNK), jnp.int32),
            pltpu.VMEM((CHUNK, 64), jnp.float32),
            pltpu.VMEM((CHUNK, 64), jnp.float32),
            pltpu.SemaphoreType.DMA,
            pltpu.SemaphoreType.DMA,
            pltpu.VMEM_SHARED((npad, 64), jnp.float32),
            pltpu.VMEM_SHARED((npad, 64), jnp.float32),
        ],
        compiler_params=_SC_PARAMS,
    )


def _enc_body(x_ref, w1_ref, b1_ref, w2_ref, d16_ref, u_ref, c64_ref):
    h = jnp.dot(x_ref[...], w1_ref[...],
                preferred_element_type=jnp.float32) + b1_ref[...]
    h = jnp.maximum(h, 0.0)
    g = jnp.dot(h, w2_ref[...], preferred_element_type=jnp.float32)
    deg = d16_ref[0, :, 0:1] + d16_ref[1, :, 0:1] + 1.0
    u_ref[...] = g * (1.0 / jnp.sqrt(deg))
    c64_ref[...] = jnp.broadcast_to(1.0 / deg, g.shape)


def _comb_body(acoef, bcoef, p_ref, c_ref, b_ref, o_ref):
    t = p_ref[0] + p_ref[1]
    o_ref[...] = acoef * c_ref[...] * t + bcoef * b_ref[...]


def _fin_body(p_ref, c_ref, m0_ref, b2_ref, o_ref):
    c = c_ref[...]
    m = 0.9 * c * (p_ref[0] + p_ref[1]) + 0.1 * m0_ref[...]
    z = m * jnp.sqrt(1.0 / c)
    a = z + b2_ref[...]
    mx = jnp.max(a, axis=1, keepdims=True)
    e = jnp.exp(a - mx)
    s = jnp.sum(e, axis=1, keepdims=True)
    o_ref[...] = (a - mx) - jnp.log(s)


@jax.jit
def kernel(x, edge_index, W1, b1, W2, b2):
    n, f_in = x.shape
    hid = W1.shape[1]
    cls = W2.shape[1]
    e = edge_index.shape[1]

    nw = NC * NT
    per_w = -(-e // (nw * 2 * CHUNK)) * 2 * CHUNK
    ep = per_w * nw
    nch = per_w // CHUNK
    npad = -(-n // (NT * CHUNK)) * NT * CHUNK

    src = edge_index[0]
    dst = edge_index[1]
    pad = ep - e
    srcp = jnp.concatenate([src, jnp.zeros((pad,), jnp.int32)])
    dstp = jnp.concatenate([dst, jnp.full((pad,), npad - 1, jnp.int32)])
    src4 = srcp.reshape(NC, NT, nch, CHUNK)
    dst4 = dstp.reshape(NC, NT, nch, CHUNK)

    deg16 = _make_deg_kernel(npad, nch)(dst4)

    bn = 400
    grid = (n // bn,)
    u, c64 = pl.pallas_call(
        _enc_body,
        grid=grid,
        in_specs=[
            pl.BlockSpec((bn, f_in), lambda i: (i, 0)),
            pl.BlockSpec((f_in, hid), lambda i: (0, 0)),
            pl.BlockSpec((1, hid), lambda i: (0, 0)),
            pl.BlockSpec((hid, cls), lambda i: (0, 0)),
            pl.BlockSpec((NC, bn, L), lambda i: (0, i, 0)),
        ],
        out_specs=[
            pl.BlockSpec((bn, cls), lambda i: (i, 0)),
            pl.BlockSpec((bn, cls), lambda i: (i, 0)),
        ],
        out_shape=[
            jax.ShapeDtypeStruct((n, cls), jnp.float32),
            jax.ShapeDtypeStruct((n, cls), jnp.float32),
        ],
    )(x, W1, b1.reshape(1, hid), W2, deg16)

    u = jnp.pad(u, ((0, npad - n), (0, 0)))
    c64 = jnp.pad(c64, ((0, npad - n), (0, 0)))

    prop = _make_prop_kernel(npad, nch)

    bnp = 512
    gridp = (npad // bnp,)

    def comb(acoef, bcoef, p, base):
        return pl.pallas_call(
            functools.partial(_comb_body, acoef, bcoef),
            grid=gridp,
            in_specs=[
                pl.BlockSpec((NC, bnp, cls), lambda i: (0, i, 0)),
                pl.BlockSpec((bnp, cls), lambda i: (i, 0)),
                pl.BlockSpec((bnp, cls), lambda i: (i, 0)),
            ],
            out_specs=pl.BlockSpec((bnp, cls), lambda i: (i, 0)),
            out_shape=jax.ShapeDtypeStruct((npad, cls), jnp.float32),
        )(p, c64, base)

    p = prop(u, src4, dst4)
    m0 = comb(-1.0 / 3.0, 1.0, p, u)
    m = m0
    for _ in range(9):
        p = prop(m, src4, dst4)
        m = comb(0.9, 0.1, p, m0)
    p = prop(m, src4, dst4)

    out = pl.pallas_call(
        _fin_body,
        grid=grid,
        in_specs=[
            pl.BlockSpec((NC, bn, cls), lambda i: (0, i, 0)),
            pl.BlockSpec((bn, cls), lambda i: (i, 0)),
            pl.BlockSpec((bn, cls), lambda i: (i, 0)),
            pl.BlockSpec((1, cls), lambda i: (0, 0)),
        ],
        out_specs=pl.BlockSpec((bn, cls), lambda i: (i, 0)),
        out_shape=jax.ShapeDtypeStruct((n, cls), jnp.float32),
    )(p, c64, m0, b2.reshape(1, cls))
    return out
